# Initial kernel scaffold; baseline (speedup 1.0000x reference)
import functools

import jax
import jax.numpy as jnp
from jax import lax
from jax.experimental import pallas as pl
from jax.experimental.pallas import tpu as pltpu
from jax.experimental.pallas import tpu_sc as plsc

EMB = 256
N = 100000
NTYPES = 64
L = 16
G = N // L
NC, NS = 2, 16
NW = NC * NS
CH_G = 8
CH = CH_G * L


def _process_group(nm_ref, table_ref, obuf_ref, go):
    iota = lax.iota(jnp.int32, L)
    rows_rel = go * L + iota
    ones = jnp.ones((L,), jnp.int32)
    types = plsc.load_gather(nm_ref, [rows_rel, ones])
    valid = types >= 0
    tsafe = jnp.maximum(types, 0)

    def col_body(jo, carry):
        for ji in range(L):
            c = jo * L + ji
            cvec = jnp.full((L,), c, jnp.int32)
            vec = plsc.load_gather(table_ref, [tsafe, cvec])
            plsc.store_scatter(obuf_ref, [rows_rel, cvec], vec, mask=valid)
        return carry

    lax.fori_loop(0, EMB // L, col_body, 0)


@functools.partial(
    pl.kernel,
    out_type=jax.ShapeDtypeStruct((N, EMB), jnp.float32),
    mesh=plsc.VectorSubcoreMesh(core_axis_name="c", subcore_axis_name="s"),
    scratch_types=[
        pltpu.VMEM((NTYPES, EMB), jnp.float32),
        pltpu.VMEM((CH, EMB), jnp.float32),
        pltpu.VMEM((CH, 2), jnp.int32),
    ],
)
def _encode(nm_hbm, init_hbm, emb_hbm, out_hbm, table_v, obuf_v, nmbuf_v):
    cid = lax.axis_index("c")
    sid = lax.axis_index("s")
    wid = sid * NC + cid
    g0 = (G * wid) // NW
    g1 = (G * (wid + 1)) // NW
    ng = g1 - g0
    nf = ng // CH_G
    nt = ng - nf * CH_G
    row0 = g0 * L

    pltpu.sync_copy(emb_hbm, table_v)

    def chunk_body(i, carry):
        r = row0 + i * CH
        pltpu.sync_copy(init_hbm.at[pl.ds(r, CH)], obuf_v)
        pltpu.sync_copy(nm_hbm.at[pl.ds(r, CH)], nmbuf_v)

        def grp(go, c2):
            _process_group(nmbuf_v, table_v, obuf_v, go)
            return c2

        lax.fori_loop(0, CH_G, grp, 0)
        pltpu.sync_copy(obuf_v, out_hbm.at[pl.ds(r, CH)])
        return carry

    lax.fori_loop(0, nf, chunk_body, 0)

    def tail_body(j, carry):
        r = row0 + nf * CH + j * L
        pltpu.sync_copy(init_hbm.at[pl.ds(r, L)], obuf_v.at[pl.ds(0, L)])
        pltpu.sync_copy(nm_hbm.at[pl.ds(r, L)], nmbuf_v.at[pl.ds(0, L)])
        _process_group(nmbuf_v, table_v, obuf_v, 0)
        pltpu.sync_copy(obuf_v.at[pl.ds(0, L)], out_hbm.at[pl.ds(r, L)])
        return carry

    lax.fori_loop(0, nt, tail_body, 0)


def kernel(node_mapping, init_embs, node_embs):
    return _encode(node_mapping, init_embs, node_embs)


# speedup vs baseline: 1.4028x; 1.4028x over previous
import functools

import jax
import jax.numpy as jnp
from jax import lax
from jax.experimental import pallas as pl
from jax.experimental.pallas import tpu as pltpu
from jax.experimental.pallas import tpu_sc as plsc

EMB = 256
N = 100000
NTYPES = 64
L = 16
G = N // L
NC, NS = 2, 16
NW = NC * NS
CH_G = 8
CH = CH_G * L


def _process_group(nm_ref, table_ref, obuf_ref, go):
    iota = lax.iota(jnp.int32, L)
    rows_rel = go * L + iota
    types = plsc.load_gather(nm_ref, [rows_rel * 2 + 1])
    valid = types >= 0
    tsafe = jnp.maximum(types, 0)
    tbase = tsafe * EMB
    sbase = rows_rel * EMB

    def col_body(jo, carry):
        ig, is_ = carry
        for _ in range(L):
            vec = plsc.load_gather(table_ref, [ig])
            plsc.store_scatter(obuf_ref, [is_], vec, mask=valid)
            ig = ig + 1
            is_ = is_ + 1
        return ig, is_

    lax.fori_loop(0, EMB // L, col_body, (tbase, sbase))


@functools.partial(
    pl.kernel,
    out_type=jax.ShapeDtypeStruct((N * EMB,), jnp.float32),
    mesh=plsc.VectorSubcoreMesh(core_axis_name="c", subcore_axis_name="s"),
    scratch_types=[
        pltpu.VMEM((NTYPES * EMB,), jnp.float32),
        pltpu.VMEM((CH * EMB,), jnp.float32),
        pltpu.VMEM((CH * 2,), jnp.int32),
    ],
    compiler_params=pltpu.CompilerParams(needs_layout_passes=False),
)
def _encode(nm_hbm, init_hbm, emb_hbm, out_hbm, table_v, obuf_v, nmbuf_v):
    cid = lax.axis_index("c")
    sid = lax.axis_index("s")
    wid = sid * NC + cid
    g0 = (G * wid) // NW
    g1 = (G * (wid + 1)) // NW
    ng = g1 - g0
    nf = ng // CH_G
    nt = ng - nf * CH_G
    row0 = g0 * L

    pltpu.sync_copy(emb_hbm, table_v)

    def chunk_body(i, carry):
        r = row0 + i * CH
        pltpu.sync_copy(init_hbm.at[pl.ds(r * EMB, CH * EMB)], obuf_v)
        pltpu.sync_copy(nm_hbm.at[pl.ds(r * 2, CH * 2)], nmbuf_v)

        def grp(go, c2):
            _process_group(nmbuf_v, table_v, obuf_v, go)
            return c2

        lax.fori_loop(0, CH_G, grp, 0)
        pltpu.sync_copy(obuf_v, out_hbm.at[pl.ds(r * EMB, CH * EMB)])
        return carry

    lax.fori_loop(0, nf, chunk_body, 0)

    def tail_body(j, carry):
        r = row0 + nf * CH + j * L
        pltpu.sync_copy(init_hbm.at[pl.ds(r * EMB, L * EMB)],
                        obuf_v.at[pl.ds(0, L * EMB)])
        pltpu.sync_copy(nm_hbm.at[pl.ds(r * 2, L * 2)],
                        nmbuf_v.at[pl.ds(0, L * 2)])
        _process_group(nmbuf_v, table_v, obuf_v, 0)
        pltpu.sync_copy(obuf_v.at[pl.ds(0, L * EMB)],
                        out_hbm.at[pl.ds(r * EMB, L * EMB)])
        return carry

    lax.fori_loop(0, nt, tail_body, 0)


def kernel(node_mapping, init_embs, node_embs):
    out_flat = _encode(node_mapping.reshape(-1), init_embs.reshape(-1),
                       node_embs.reshape(-1))
    return out_flat.reshape(N, EMB)


# speedup vs baseline: 1.4994x; 1.0689x over previous
import functools

import jax
import jax.numpy as jnp
from jax import lax
from jax.experimental import pallas as pl
from jax.experimental.pallas import tpu as pltpu
from jax.experimental.pallas import tpu_sc as plsc

EMB = 256
N = 100000
NTYPES = 64
L = 16
G = N // L
NC, NS = 2, 16
NW = NC * NS
CH_G = 8
CH = CH_G * L
NBUF = 3
NF = 195 // CH_G


def _process_group(nm_ref, table_ref, obuf_ref, go):
    iota = lax.iota(jnp.int32, L)
    rows_rel = go * L + iota
    types = plsc.load_gather(nm_ref, [rows_rel * 2 + 1])
    valid = types >= 0
    tsafe = jnp.maximum(types, 0)

    def col_body(jo, carry):
        ig, is_ = carry
        for _ in range(L):
            vec = plsc.load_gather(table_ref, [ig])
            plsc.store_scatter(obuf_ref, [is_], vec, mask=valid)
            ig = ig + 1
            is_ = is_ + 1
        return ig, is_

    lax.fori_loop(0, EMB // L, col_body, (tsafe * EMB, rows_rel * EMB))


@functools.partial(
    pl.kernel,
    out_type=jax.ShapeDtypeStruct((N * EMB,), jnp.float32),
    mesh=plsc.VectorSubcoreMesh(core_axis_name="c", subcore_axis_name="s"),
    scratch_types=[
        pltpu.VMEM((NTYPES * EMB,), jnp.float32),
        [pltpu.VMEM((CH * EMB,), jnp.float32)] * NBUF,
        [pltpu.VMEM((CH * 2,), jnp.int32)] * NBUF,
        [pltpu.SemaphoreType.DMA] * NBUF,
        [pltpu.SemaphoreType.DMA] * NBUF,
        pltpu.SemaphoreType.DMA,
    ],
    compiler_params=pltpu.CompilerParams(needs_layout_passes=False),
)
def _encode(nm_hbm, init_hbm, emb_hbm, out_hbm, table_v, obufs, nmbufs,
            isems, osems, tsem):
    cid = lax.axis_index("c")
    sid = lax.axis_index("s")
    wid = sid * NC + cid
    g0 = (G * wid) // NW
    g1 = (G * (wid + 1)) // NW
    nt = (g1 - g0) - NF * CH_G
    row0 = g0 * L

    def in_copies(i, b):
        r = row0 + i * CH
        return (
            pltpu.make_async_copy(
                init_hbm.at[pl.ds(r * EMB, CH * EMB)], obufs[b], isems[b]),
            pltpu.make_async_copy(
                nm_hbm.at[pl.ds(r * 2, CH * 2)], nmbufs[b], isems[b]),
        )

    def out_copy(i, b):
        r = row0 + i * CH
        return pltpu.make_async_copy(
            obufs[b], out_hbm.at[pl.ds(r * EMB, CH * EMB)], osems[b])

    def compute(b):
        def grp(go, c2):
            _process_group(nmbufs[b], table_v, obufs[b], go)
            return c2
        lax.fori_loop(0, CH_G, grp, 0)

    tcopy = pltpu.make_async_copy(emb_hbm, table_v, tsem)
    tcopy.start()
    for d in in_copies(0, 0):
        d.start()
    for d in in_copies(1, 1):
        d.start()
    tcopy.wait()

    for i in range(NF):
        b = i % NBUF
        if i + 1 < NF:
            b2 = (i + 1) % NBUF
            if i - 2 >= 0:
                out_copy(i - 2, b2).wait()
            for d in in_copies(i + 1, b2):
                d.start()
        for d in in_copies(i, b):
            d.wait()
        compute(b)
        out_copy(i, b).start()

    for i in (NF - 3, NF - 2, NF - 1):
        out_copy(i, i % NBUF).wait()

    def tail_body(j, carry):
        r = row0 + NF * CH + j * L
        pltpu.sync_copy(init_hbm.at[pl.ds(r * EMB, L * EMB)],
                        obufs[0].at[pl.ds(0, L * EMB)])
        pltpu.sync_copy(nm_hbm.at[pl.ds(r * 2, L * 2)],
                        nmbufs[0].at[pl.ds(0, L * 2)])
        _process_group(nmbufs[0], table_v, obufs[0], 0)
        pltpu.sync_copy(obufs[0].at[pl.ds(0, L * EMB)],
                        out_hbm.at[pl.ds(r * EMB, L * EMB)])
        return carry

    lax.fori_loop(0, nt, tail_body, 0)


def kernel(node_mapping, init_embs, node_embs):
    out_flat = _encode(node_mapping.reshape(-1), init_embs.reshape(-1),
                       node_embs.reshape(-1))
    return out_flat.reshape(N, EMB)


# speedup vs baseline: 2.9159x; 1.9447x over previous
import functools

import jax
import jax.numpy as jnp
from jax import lax
from jax.experimental import pallas as pl
from jax.experimental.pallas import tpu as pltpu
from jax.experimental.pallas import tpu_sc as plsc

EMB = 256
N = 100000
NTYPES = 64
L = 16
G = N // L
NC, NS = 2, 16
NW = NC * NS
CH_G = 8
CH = CH_G * L
NBUF = 3
NF = 195 // CH_G


def _process_group(nm_ref, table_ref, obuf_ref, go):
    iota = lax.iota(jnp.int32, L)
    rows_rel = go * L + iota
    types = plsc.load_gather(nm_ref, [rows_rel * 2 + 1])
    valid = types >= 0
    tbase = jnp.maximum(types, 0) * EMB
    sbase = rows_rel * EMB

    @plsc.parallel_loop(0, EMB, unroll=16)
    def col_body(c):
        vec = plsc.load_gather(table_ref, [tbase + c])
        plsc.store_scatter(obuf_ref, [sbase + c], vec, mask=valid)


@functools.partial(
    pl.kernel,
    out_type=jax.ShapeDtypeStruct((N * EMB,), jnp.float32),
    mesh=plsc.VectorSubcoreMesh(core_axis_name="c", subcore_axis_name="s"),
    scratch_types=[
        pltpu.VMEM((NTYPES * EMB,), jnp.float32),
        [pltpu.VMEM((CH * EMB,), jnp.float32)] * NBUF,
        [pltpu.VMEM((CH * 2,), jnp.int32)] * NBUF,
        [pltpu.SemaphoreType.DMA] * NBUF,
        [pltpu.SemaphoreType.DMA] * NBUF,
        pltpu.SemaphoreType.DMA,
    ],
    compiler_params=pltpu.CompilerParams(needs_layout_passes=False),
)
def _encode(nm_hbm, init_hbm, emb_hbm, out_hbm, table_v, obufs, nmbufs,
            isems, osems, tsem):
    cid = lax.axis_index("c")
    sid = lax.axis_index("s")
    wid = sid * NC + cid
    g0 = (G * wid) // NW
    g1 = (G * (wid + 1)) // NW
    nt = (g1 - g0) - NF * CH_G
    row0 = g0 * L

    def in_copies(i, b):
        r = row0 + i * CH
        return (
            pltpu.make_async_copy(
                init_hbm.at[pl.ds(r * EMB, CH * EMB)], obufs[b], isems[b]),
            pltpu.make_async_copy(
                nm_hbm.at[pl.ds(r * 2, CH * 2)], nmbufs[b], isems[b]),
        )

    def out_copy(i, b):
        r = row0 + i * CH
        return pltpu.make_async_copy(
            obufs[b], out_hbm.at[pl.ds(r * EMB, CH * EMB)], osems[b])

    def compute(b):
        def grp(go, c2):
            _process_group(nmbufs[b], table_v, obufs[b], go)
            return c2
        lax.fori_loop(0, CH_G, grp, 0)

    tcopy = pltpu.make_async_copy(emb_hbm, table_v, tsem)
    tcopy.start()
    for d in in_copies(0, 0):
        d.start()
    tcopy.wait()

    for i in range(NF):
        b = i % NBUF
        if i + 1 < NF:
            b2 = (i + 1) % NBUF
            if i - 2 >= 0:
                out_copy(i - 2, b2).wait()
            for d in in_copies(i + 1, b2):
                d.start()
        for d in in_copies(i, b):
            d.wait()
        compute(b)
        out_copy(i, b).start()

    for i in (NF - 3, NF - 2, NF - 1):
        out_copy(i, i % NBUF).wait()

    def tail_body(j, carry):
        r = row0 + NF * CH + j * L
        pltpu.sync_copy(init_hbm.at[pl.ds(r * EMB, L * EMB)],
                        obufs[0].at[pl.ds(0, L * EMB)])
        pltpu.sync_copy(nm_hbm.at[pl.ds(r * 2, L * 2)],
                        nmbufs[0].at[pl.ds(0, L * 2)])
        _process_group(nmbufs[0], table_v, obufs[0], 0)
        pltpu.sync_copy(obufs[0].at[pl.ds(0, L * EMB)],
                        out_hbm.at[pl.ds(r * EMB, L * EMB)])
        return carry

    lax.fori_loop(0, nt, tail_body, 0)


def kernel(node_mapping, init_embs, node_embs):
    out_flat = _encode(node_mapping.reshape(-1), init_embs.reshape(-1),
                       node_embs.reshape(-1))
    return out_flat.reshape(N, EMB)


# speedup vs baseline: 5.2948x; 1.8158x over previous
import functools

import jax
import jax.numpy as jnp
from jax import lax
from jax.experimental import pallas as pl
from jax.experimental.pallas import tpu as pltpu
from jax.experimental.pallas import tpu_sc as plsc

EMB = 256
N = 100000
NTYPES = 64
L = 16
G = N // L
NC, NS = 2, 16
NW = NC * NS
CH_G = 8
CH = CH_G * L
NBUF = 3
NF = 195 // CH_G


def _process_group(nm_ref, table_ref, obuf_ref, go):
    iota = lax.iota(jnp.int32, L)
    rows_rel = go * L + iota
    types = plsc.load_gather(nm_ref, [rows_rel * 2 + 1])
    valid = types >= 0
    tbase = jnp.maximum(types, 0) * EMB
    sbase = rows_rel * EMB

    @plsc.parallel_loop(0, EMB, unroll=16)
    def col_body(c):
        coff = (iota + c) & (EMB - 1)
        vec = plsc.load_gather(table_ref, [tbase + coff])
        plsc.store_scatter(obuf_ref, [sbase + coff], vec, mask=valid)


@functools.partial(
    pl.kernel,
    out_type=jax.ShapeDtypeStruct((N * EMB,), jnp.float32),
    mesh=plsc.VectorSubcoreMesh(core_axis_name="c", subcore_axis_name="s"),
    scratch_types=[
        pltpu.VMEM((NTYPES * EMB,), jnp.float32),
        [pltpu.VMEM((CH * EMB,), jnp.float32)] * NBUF,
        [pltpu.VMEM((CH * 2,), jnp.int32)] * NBUF,
        [pltpu.SemaphoreType.DMA] * NBUF,
        [pltpu.SemaphoreType.DMA] * NBUF,
        pltpu.SemaphoreType.DMA,
    ],
    compiler_params=pltpu.CompilerParams(needs_layout_passes=False),
)
def _encode(nm_hbm, init_hbm, emb_hbm, out_hbm, table_v, obufs, nmbufs,
            isems, osems, tsem):
    cid = lax.axis_index("c")
    sid = lax.axis_index("s")
    wid = sid * NC + cid
    g0 = (G * wid) // NW
    g1 = (G * (wid + 1)) // NW
    nt = (g1 - g0) - NF * CH_G
    row0 = g0 * L

    def in_copies(i, b):
        r = row0 + i * CH
        return (
            pltpu.make_async_copy(
                init_hbm.at[pl.ds(r * EMB, CH * EMB)], obufs[b], isems[b]),
            pltpu.make_async_copy(
                nm_hbm.at[pl.ds(r * 2, CH * 2)], nmbufs[b], isems[b]),
        )

    def out_copy(i, b):
        r = row0 + i * CH
        return pltpu.make_async_copy(
            obufs[b], out_hbm.at[pl.ds(r * EMB, CH * EMB)], osems[b])

    def compute(b):
        def grp(go, c2):
            _process_group(nmbufs[b], table_v, obufs[b], go)
            return c2
        lax.fori_loop(0, CH_G, grp, 0)

    tcopy = pltpu.make_async_copy(emb_hbm, table_v, tsem)
    tcopy.start()
    for d in in_copies(0, 0):
        d.start()
    tcopy.wait()

    for i in range(NF):
        b = i % NBUF
        if i + 1 < NF:
            b2 = (i + 1) % NBUF
            if i - 2 >= 0:
                out_copy(i - 2, b2).wait()
            for d in in_copies(i + 1, b2):
                d.start()
        for d in in_copies(i, b):
            d.wait()
        compute(b)
        out_copy(i, b).start()

    for i in (NF - 3, NF - 2, NF - 1):
        out_copy(i, i % NBUF).wait()

    def tail_body(j, carry):
        r = row0 + NF * CH + j * L
        pltpu.sync_copy(init_hbm.at[pl.ds(r * EMB, L * EMB)],
                        obufs[0].at[pl.ds(0, L * EMB)])
        pltpu.sync_copy(nm_hbm.at[pl.ds(r * 2, L * 2)],
                        nmbufs[0].at[pl.ds(0, L * 2)])
        _process_group(nmbufs[0], table_v, obufs[0], 0)
        pltpu.sync_copy(obufs[0].at[pl.ds(0, L * EMB)],
                        out_hbm.at[pl.ds(r * EMB, L * EMB)])
        return carry

    lax.fori_loop(0, nt, tail_body, 0)


def kernel(node_mapping, init_embs, node_embs):
    out_flat = _encode(node_mapping.reshape(-1), init_embs.reshape(-1),
                       node_embs.reshape(-1))
    return out_flat.reshape(N, EMB)


# speedup vs baseline: 8.0170x; 1.5141x over previous
import functools

import jax
import jax.numpy as jnp
from jax import lax
from jax.experimental import pallas as pl
from jax.experimental.pallas import tpu as pltpu
from jax.experimental.pallas import tpu_sc as plsc

EMB = 256
N = 100000
NTYPES = 64
L = 16
G = N // L
NC, NS = 2, 16
NW = NC * NS
CH_G = 8
CH = CH_G * L
NBUF = 3
NF = 195 // CH_G


def _fill_group(tbuf_ref, table_ref, obuf_ref, go):
    iota = lax.iota(jnp.int32, L)
    types = tbuf_ref[pl.ds(go * L, L)]
    valid = types >= 0
    tsafe = jnp.maximum(types, 0)
    tbase = tsafe * EMB
    sbase = (go * L + iota) * EMB

    @plsc.parallel_loop(0, EMB, unroll=16)
    def col_body(c):
        coff = (iota + c) & (EMB - 1)
        vec = plsc.load_gather(table_ref, [tbase + coff])
        plsc.store_scatter(obuf_ref, [sbase + coff], vec, mask=valid)

    return types


@functools.partial(
    pl.kernel,
    out_type=jax.ShapeDtypeStruct((N * EMB,), jnp.float32),
    mesh=plsc.VectorSubcoreMesh(core_axis_name="c", subcore_axis_name="s"),
    scratch_types=[
        pltpu.VMEM((NTYPES * EMB,), jnp.float32),
        [pltpu.VMEM((CH * EMB,), jnp.float32)] * NBUF,
        [pltpu.VMEM((CH,), jnp.int32)] * NBUF,
        [pltpu.SemaphoreType.DMA] * NBUF,
        [pltpu.SemaphoreType.DMA] * NBUF,
        pltpu.SemaphoreType.DMA,
        pltpu.SemaphoreType.DMA,
    ],
    compiler_params=pltpu.CompilerParams(needs_layout_passes=False),
)
def _encode(types_hbm, init_hbm, emb_hbm, out_hbm, table_v, obufs, tbufs,
            isems, osems, tsem, vsem):
    cid = lax.axis_index("c")
    sid = lax.axis_index("s")
    wid = sid * NC + cid
    g0 = (G * wid) // NW
    g1 = (G * (wid + 1)) // NW
    nt = (g1 - g0) - NF * CH_G
    row0 = g0 * L
    iota = lax.iota(jnp.int32, L)

    def in_copy(i, b):
        r = row0 + i * CH
        return pltpu.make_async_copy(
            types_hbm.at[pl.ds(r, CH)], tbufs[b], isems[b])

    def out_copy(i, b):
        r = row0 + i * CH
        return pltpu.make_async_copy(
            obufs[b], out_hbm.at[pl.ds(r * EMB, CH * EMB)], osems[b])

    def invalid_dma(grow, rr, b):
        return pltpu.make_async_copy(
            init_hbm.at[grow], obufs[b].at[pl.ds(rr * EMB, EMB)], vsem)

    def compute(i, b):
        def grp(go, k):
            types = _fill_group(tbufs[b], table_v, obufs[b], go)
            inv = types < 0
            kg = jnp.max(plsc.all_reduce_population_count(inv))

            def issue(j, m):
                lane = plsc.all_reduce_ffs(m)
                rr = go * L + jnp.max(lane)
                invalid_dma(row0 + i * CH + rr, rr, b).start()
                return m & (iota != lane)

            lax.fori_loop(0, kg, issue, inv)
            return k + kg
        return lax.fori_loop(0, CH_G, grp, jnp.int32(0))

    def drain_invalid(k, b):
        def w(j, c2):
            invalid_dma(0, 0, b).wait()
            return c2
        lax.fori_loop(0, k, w, 0)

    tcopy = pltpu.make_async_copy(emb_hbm, table_v, tsem)
    tcopy.start()
    in_copy(0, 0).start()
    tcopy.wait()

    for i in range(NF):
        b = i % NBUF
        if i + 1 < NF:
            b2 = (i + 1) % NBUF
            if i - 2 >= 0:
                out_copy(i - 2, b2).wait()
            in_copy(i + 1, b2).start()
        in_copy(i, b).wait()
        k = compute(i, b)
        drain_invalid(k, b)
        out_copy(i, b).start()

    for i in (NF - 3, NF - 2, NF - 1):
        out_copy(i, i % NBUF).wait()

    def tail_body(j, carry):
        r = row0 + NF * CH + j * L
        pltpu.sync_copy(types_hbm.at[pl.ds(r, L)], tbufs[0].at[pl.ds(0, L)])
        types = _fill_group(tbufs[0], table_v, obufs[0], 0)
        inv = types < 0
        kg = jnp.max(plsc.all_reduce_population_count(inv))

        def issue(jj, m):
            lane = plsc.all_reduce_ffs(m)
            ls = jnp.max(lane)
            invalid_dma(r + ls, ls, 0).start()
            return m & (iota != lane)

        lax.fori_loop(0, kg, issue, inv)
        drain_invalid(kg, 0)
        pltpu.sync_copy(obufs[0].at[pl.ds(0, L * EMB)],
                        out_hbm.at[pl.ds(r * EMB, L * EMB)])
        return carry

    lax.fori_loop(0, nt, tail_body, 0)


def kernel(node_mapping, init_embs, node_embs):
    out_flat = _encode(node_mapping[:, 1], init_embs, node_embs.reshape(-1))
    return out_flat.reshape(N, EMB)


# speedup vs baseline: 12.1935x; 1.5210x over previous
import functools

import jax
import jax.numpy as jnp
from jax import lax
from jax.experimental import pallas as pl
from jax.experimental.pallas import tpu as pltpu
from jax.experimental.pallas import tpu_sc as plsc

EMB = 256
N = 100000
NTYPES = 64
L = 16
G = N // L
NC, NS = 2, 16
NW = NC * NS
CH_G = 8
CH = CH_G * L
NBUF = 3
NF = 195 // CH_G


def _fill_group(tbuf_ref, table_ref, obuf_ref, go):
    iota = lax.iota(jnp.int32, L)
    types = tbuf_ref[pl.ds(go * L, L)]
    valid = types >= 0
    tbase = jnp.maximum(types, 0) * EMB
    rows = go * L + iota

    @plsc.parallel_loop(0, EMB, unroll=16)
    def col_body(c):
        coff = (iota + c) & (EMB - 1)
        vec = plsc.load_gather(table_ref, [tbase + coff])
        plsc.store_scatter(obuf_ref, [rows, coff], vec, mask=valid)

    return types


@functools.partial(
    pl.kernel,
    out_type=jax.ShapeDtypeStruct((N, EMB), jnp.float32),
    mesh=plsc.VectorSubcoreMesh(core_axis_name="c", subcore_axis_name="s"),
    scratch_types=[
        pltpu.VMEM((NTYPES * EMB,), jnp.float32),
        [pltpu.VMEM((CH, EMB), jnp.float32)] * NBUF,
        [pltpu.VMEM((CH,), jnp.int32)] * NBUF,
        [pltpu.SemaphoreType.DMA] * NBUF,
        [pltpu.SemaphoreType.DMA] * NBUF,
        pltpu.SemaphoreType.DMA,
        pltpu.SemaphoreType.DMA,
    ],
    compiler_params=pltpu.CompilerParams(needs_layout_passes=False),
)
def _encode(types_hbm, init_hbm, emb_hbm, out_hbm, table_v, obufs, tbufs,
            isems, osems, tsem, vsem):
    cid = lax.axis_index("c")
    sid = lax.axis_index("s")
    wid = sid * NC + cid
    g0 = (G * wid) // NW
    g1 = (G * (wid + 1)) // NW
    nt = (g1 - g0) - NF * CH_G
    row0 = g0 * L
    iota = lax.iota(jnp.int32, L)

    def in_copy(i, b):
        r = row0 + i * CH
        return pltpu.make_async_copy(
            types_hbm.at[pl.ds(r, CH)], tbufs[b], isems[b])

    def out_copy(i, b):
        r = row0 + i * CH
        return pltpu.make_async_copy(
            obufs[b], out_hbm.at[pl.ds(r, CH)], osems[b])

    def invalid_dma(grow, rr, b):
        return pltpu.make_async_copy(
            init_hbm.at[grow], obufs[b].at[rr], vsem)

    def compute(i, b):
        def grp(go, k):
            types = _fill_group(tbufs[b], table_v, obufs[b], go)
            inv = types < 0
            kg = jnp.max(plsc.all_reduce_population_count(inv))

            def issue(j, m):
                lane = plsc.all_reduce_ffs(m)
                rr = go * L + jnp.max(lane)
                invalid_dma(row0 + i * CH + rr, rr, b).start()
                return m & (iota != lane)

            lax.fori_loop(0, kg, issue, inv)
            return k + kg
        return lax.fori_loop(0, CH_G, grp, jnp.int32(0))

    def drain_invalid(k, b):
        def w(j, c2):
            invalid_dma(0, 0, b).wait()
            return c2
        lax.fori_loop(0, k, w, 0)

    tcopy = pltpu.make_async_copy(emb_hbm, table_v, tsem)
    tcopy.start()
    in_copy(0, 0).start()
    tcopy.wait()

    for i in range(NF):
        b = i % NBUF
        if i + 1 < NF:
            b2 = (i + 1) % NBUF
            if i - 2 >= 0:
                out_copy(i - 2, b2).wait()
            in_copy(i + 1, b2).start()
        in_copy(i, b).wait()
        k = compute(i, b)
        drain_invalid(k, b)
        out_copy(i, b).start()

    for i in (NF - 3, NF - 2, NF - 1):
        out_copy(i, i % NBUF).wait()

    def tail_body(j, carry):
        r = row0 + NF * CH + j * L
        pltpu.sync_copy(types_hbm.at[pl.ds(r, L)], tbufs[0].at[pl.ds(0, L)])
        types = _fill_group(tbufs[0], table_v, obufs[0], 0)
        inv = types < 0
        kg = jnp.max(plsc.all_reduce_population_count(inv))

        def issue(jj, m):
            lane = plsc.all_reduce_ffs(m)
            ls = jnp.max(lane)
            invalid_dma(r + ls, ls, 0).start()
            return m & (iota != lane)

        lax.fori_loop(0, kg, issue, inv)
        drain_invalid(kg, 0)
        pltpu.sync_copy(obufs[0].at[pl.ds(0, L)],
                        out_hbm.at[pl.ds(r, L)])
        return carry

    lax.fori_loop(0, nt, tail_body, 0)


def kernel(node_mapping, init_embs, node_embs):
    return _encode(node_mapping[:, 1], init_embs, node_embs.reshape(-1))
